# Initial kernel scaffold; baseline (speedup 1.0000x reference)
import functools

import jax
import jax.numpy as jnp
from jax import lax
from jax.experimental import pallas as pl
from jax.experimental.pallas import tpu as pltpu
from jax.experimental.pallas import tpu_sc as plsc

N = 10000
D = 128
NC = 2
NS = 16
NW = NC * NS
B = 128
ROWS_PER_TILE = N // NS
ZR = 125


def _invnorm_body(x_ref, o_ref):
    xb = x_ref[...]
    ss = jnp.sum(xb * xb, axis=1, keepdims=True)
    inv = lax.rsqrt(jnp.maximum(ss, 1e-24))
    o_ref[...] = jnp.broadcast_to(inv, xb.shape)


def _final_body(a0_ref, a1_ref, dt_ref, o_ref):
    den = jnp.sum(dt_ref[...], axis=1, keepdims=True)
    s = a0_ref[...] + a1_ref[...]
    o_ref[...] = jnp.maximum(s / jnp.maximum(den, 1e-16), 0.0)


def _make_edge_kernel(e_full, n_blk, cpt):
    mesh = plsc.VectorSubcoreMesh(
        core_axis_name="c", subcore_axis_name="s", num_cores=NC, num_subcores=NS
    )

    @functools.partial(
        pl.kernel,
        out_type=(
            jax.ShapeDtypeStruct((NC, N, D), jnp.float32),
            jax.ShapeDtypeStruct((NW, N), jnp.float32),
        ),
        mesh=mesh,
        scratch_types=[
            pltpu.VMEM((N,), jnp.float32),
            pltpu.VMEM((N,), jnp.float32),
            pltpu.VMEM((1, B), jnp.int32),
            pltpu.VMEM((1, B), jnp.int32),
            pltpu.VMEM((B, D), jnp.float32),
            pltpu.VMEM((B, D), jnp.float32),
            pltpu.VMEM((B,), jnp.float32),
            pltpu.VMEM((B,), jnp.float32),
            pltpu.VMEM((16,), jnp.float32),
            pltpu.VMEM((ZR, D), jnp.float32),
            pltpu.SemaphoreType.DMA,
            pltpu.VMEM_SHARED((N, D), jnp.float32),
        ],
    )
    def edge_kernel(x_hbm, invn_hbm, src_hbm, dst_hbm, beta_hbm,
                    acc_out, den_out,
                    invn_v, denom_v, sidx, didx, srows, drows, wbuf, dots,
                    beta_v, zrows, gsem, acc_sh):
        cid = lax.axis_index("c")
        sid = lax.axis_index("s")
        wid = cid * NS + sid

        pltpu.sync_copy(invn_hbm, invn_v)
        pltpu.sync_copy(beta_hbm, beta_v)
        b = beta_v[0]

        zero16 = jnp.zeros((16,), jnp.float32)

        def zero_den(i, carry):
            denom_v[pl.ds(i * 16, 16)] = zero16
            return carry

        lax.fori_loop(0, N // 16, zero_den, 0)

        def zero_zr(i, carry):
            for s in range(8):
                zrows[i, pl.ds(s * 16, 16)] = zero16
            return carry

        lax.fori_loop(0, ZR, zero_zr, 0)

        row0 = sid * ROWS_PER_TILE
        for q in range(ROWS_PER_TILE // ZR):
            pltpu.sync_copy(zrows, acc_sh.at[pl.ds(row0 + q * ZR, ZR)])
        plsc.subcore_barrier()

        def blk(g, carry):
            base = wid * cpt + g * B
            pltpu.sync_copy(src_hbm.at[pl.ds(base, B)], sidx.at[0])
            pltpu.sync_copy(dst_hbm.at[pl.ds(base, B)], didx.at[0])
            cps = pltpu.async_copy(x_hbm.at[sidx.at[0]], srows, gsem)
            cpd = pltpu.async_copy(x_hbm.at[didx.at[0]], drows, gsem)
            cps.wait()
            cpd.wait()

            def dot_edge(e, c2):
                acc = srows[e, pl.ds(0, 16)] * drows[e, pl.ds(0, 16)]
                for s in range(1, 8):
                    sl = pl.ds(s * 16, 16)
                    acc = acc + srows[e, sl] * drows[e, sl]
                dots[e] = jnp.sum(acc)
                return c2

            lax.fori_loop(0, B, dot_edge, 0)

            def grp(k, c2):
                sl = pl.ds(k * 16, 16)
                d16 = dots[sl]
                si = sidx[0, sl]
                di = didx[0, sl]
                inv_s = plsc.load_gather(invn_v, [si])
                inv_d = plsc.load_gather(invn_v, [di])
                gid = base + k * 16 + lax.iota(jnp.int32, 16)
                w = jnp.where(gid < e_full,
                              jnp.exp(b * d16 * inv_s * inv_d),
                              0.0)
                wbuf[sl] = w
                plsc.addupdate_scatter(denom_v, [di], w)
                return c2

            lax.fori_loop(0, B // 16, grp, 0)

            def scale_edge(e, c2):
                w = wbuf[e]
                for s in range(8):
                    sl = pl.ds(s * 16, 16)
                    srows[e, sl] = srows[e, sl] * w
                return c2

            lax.fori_loop(0, B, scale_edge, 0)

            pltpu.sync_copy(srows, acc_sh.at[didx.at[0]], add=True)
            return carry

        lax.fori_loop(0, n_blk, blk, 0)

        plsc.subcore_barrier()
        pltpu.sync_copy(acc_sh.at[pl.ds(row0, ROWS_PER_TILE)],
                        acc_out.at[cid, pl.ds(row0, ROWS_PER_TILE)])
        pltpu.sync_copy(denom_v, den_out.at[wid])

    return edge_kernel


def kernel(x, edge_index, beta):
    src = edge_index[0].astype(jnp.int32)
    dst = edge_index[1].astype(jnp.int32)
    loop = jnp.arange(N, dtype=jnp.int32)
    e_full = src.shape[0] + N
    n_blk = -(-e_full // (NW * B))
    cpt = n_blk * B
    e_pad = NW * cpt
    pad = e_pad - e_full
    src_full = jnp.concatenate([src, loop, jnp.zeros((pad,), jnp.int32)])
    dst_full = jnp.concatenate([dst, loop, jnp.zeros((pad,), jnp.int32)])
    beta16 = jnp.broadcast_to(beta.astype(jnp.float32), (16,))

    grid_r = 10
    rb = N // grid_r
    invn2d = pl.pallas_call(
        _invnorm_body,
        grid=(grid_r,),
        in_specs=[pl.BlockSpec((rb, D), lambda i: (i, 0))],
        out_specs=pl.BlockSpec((rb, D), lambda i: (i, 0)),
        out_shape=jax.ShapeDtypeStruct((N, D), jnp.float32),
    )(x)
    invn = invn2d[:, 0]

    edge_fn = _make_edge_kernel(e_full, n_blk, cpt)
    acc, denp = edge_fn(x, invn, src_full, dst_full, beta16)

    out = pl.pallas_call(
        _final_body,
        grid=(grid_r,),
        in_specs=[
            pl.BlockSpec((rb, D), lambda i: (i, 0)),
            pl.BlockSpec((rb, D), lambda i: (i, 0)),
            pl.BlockSpec((rb, NW), lambda i: (i, 0)),
        ],
        out_specs=pl.BlockSpec((rb, D), lambda i: (i, 0)),
        out_shape=jax.ShapeDtypeStruct((N, D), jnp.float32),
    )(acc[0], acc[1], denp.T)
    return out


# speedup vs baseline: 9.1685x; 9.1685x over previous
import functools

import jax
import jax.numpy as jnp
from jax import lax
from jax.experimental import pallas as pl
from jax.experimental.pallas import tpu as pltpu
from jax.experimental.pallas import tpu_sc as plsc

N = 10000
D = 128
NC = 2
NS = 16
NW = NC * NS
B = 64
NP = 10240
ROWS_PER_TILE = NP // NS
DEN_R = NP // 128


def _norm_body(x_ref, xn_ref, nb_ref):
    xb = x_ref[...]
    ss = jnp.sum(xb * xb, axis=1, keepdims=True)
    nrm = jnp.maximum(jnp.sqrt(ss), 1e-12)
    xn_ref[...] = xb / nrm
    nb_ref[...] = jnp.broadcast_to(nrm, xb.shape)


def _final_body(a0_ref, a1_ref, dt_ref, o_ref):
    den = jnp.sum(dt_ref[...], axis=1, keepdims=True)
    s = a0_ref[...] + a1_ref[...]
    o_ref[...] = jnp.maximum(s / jnp.maximum(den, 1e-16), 0.0)


def _make_edge_kernel(e_full, n_blk, cpt):
    mesh = plsc.VectorSubcoreMesh(
        core_axis_name="c", subcore_axis_name="s", num_cores=NC, num_subcores=NS
    )

    @functools.partial(
        pl.kernel,
        out_type=(
            jax.ShapeDtypeStruct((NC, NP, D), jnp.float32),
            jax.ShapeDtypeStruct((NW, DEN_R, 128), jnp.float32),
        ),
        mesh=mesh,
        compiler_params=pltpu.CompilerParams(needs_layout_passes=False),
        scratch_types=[
            pltpu.VMEM((N,), jnp.float32),
            pltpu.VMEM((DEN_R, 128), jnp.float32),
            pltpu.VMEM((1, B), jnp.int32),
            pltpu.VMEM((1, B), jnp.int32),
            pltpu.VMEM((B, D), jnp.float32),
            pltpu.VMEM((B, D), jnp.float32),
            pltpu.VMEM((16,), jnp.float32),
            pltpu.SemaphoreType.DMA,
            pltpu.VMEM_SHARED((NP, D), jnp.float32),
        ],
    )
    def edge_kernel(xn_hbm, n_hbm, src_hbm, dst_hbm, beta_hbm,
                    acc_out, den_out,
                    ntab, denom_v, sidx, didx, srows, drows,
                    beta_v, gsem, acc_sh):
        cid = lax.axis_index("c")
        sid = lax.axis_index("s")
        wid = cid * NS + sid

        pltpu.sync_copy(n_hbm, ntab)
        pltpu.sync_copy(beta_hbm, beta_v)
        bvec = beta_v[pl.ds(0, 16)]

        zero16 = jnp.zeros((16,), jnp.float32)

        def zero_den(i, carry):
            for s in range(8):
                denom_v[i, pl.ds(s * 16, 16)] = zero16
            return carry

        lax.fori_loop(0, DEN_R, zero_den, 0)

        def zero_sr(i, carry):
            for s in range(8):
                srows[i, pl.ds(s * 16, 16)] = zero16
            return carry

        lax.fori_loop(0, B, zero_sr, 0)

        row0 = sid * ROWS_PER_TILE
        for q in range(ROWS_PER_TILE // B):
            pltpu.sync_copy(srows, acc_sh.at[pl.ds(row0 + q * B, B)])
        plsc.subcore_barrier()

        lane = lax.iota(jnp.int32, 16)

        def blk(g, carry):
            base = wid * cpt + g * B
            pltpu.sync_copy(src_hbm.at[pl.ds(base, B)], sidx.at[0])
            pltpu.sync_copy(dst_hbm.at[pl.ds(base, B)], didx.at[0])
            cps = pltpu.async_copy(xn_hbm.at[sidx.at[0]], srows, gsem)
            cpd = pltpu.async_copy(xn_hbm.at[didx.at[0]], drows, gsem)
            cps.wait()
            cpd.wait()

            def grp(k, c2):
                sl = pl.ds(k * 16, 16)
                si = sidx[0, sl]
                di = didx[0, sl]
                d16 = jnp.zeros((16,), jnp.float32)
                for j in range(16):
                    e = k * 16 + j
                    acc = srows[e, pl.ds(0, 16)] * drows[e, pl.ds(0, 16)]
                    for s in range(1, 8):
                        fsl = pl.ds(s * 16, 16)
                        acc = acc + srows[e, fsl] * drows[e, fsl]
                    d16 = jnp.where(lane == j, jnp.sum(acc), d16)
                gid = base + k * 16 + lane
                w = jnp.where(gid < e_full, jnp.exp(bvec * d16), 0.0)
                plsc.addupdate_scatter(denom_v, [di // 128, di % 128], w)
                n_s = plsc.load_gather(ntab, [si])
                wn = w * n_s
                for j in range(16):
                    e = k * 16 + j
                    wj = wn[j]
                    for s in range(8):
                        fsl = pl.ds(s * 16, 16)
                        srows[e, fsl] = srows[e, fsl] * wj
                return c2

            lax.fori_loop(0, B // 16, grp, 0)

            pltpu.sync_copy(srows, acc_sh.at[didx.at[0]], add=True)
            return carry

        lax.fori_loop(0, n_blk, blk, 0)

        plsc.subcore_barrier()
        pltpu.sync_copy(acc_sh.at[pl.ds(row0, ROWS_PER_TILE)],
                        acc_out.at[cid, pl.ds(row0, ROWS_PER_TILE)])
        pltpu.sync_copy(denom_v, den_out.at[wid])

    return edge_kernel


def kernel(x, edge_index, beta):
    src = edge_index[0].astype(jnp.int32)
    dst = edge_index[1].astype(jnp.int32)
    loop = jnp.arange(N, dtype=jnp.int32)
    e_full = src.shape[0] + N
    n_blk = -(-e_full // (NW * B))
    cpt = n_blk * B
    e_pad = NW * cpt
    pad = e_pad - e_full
    src_full = jnp.concatenate([src, loop, jnp.zeros((pad,), jnp.int32)])
    dst_full = jnp.concatenate([dst, loop, jnp.zeros((pad,), jnp.int32)])
    beta16 = jnp.broadcast_to(beta.astype(jnp.float32), (16,))

    grid_r = 10
    rb = N // grid_r
    xn, nb = pl.pallas_call(
        _norm_body,
        grid=(grid_r,),
        in_specs=[pl.BlockSpec((rb, D), lambda i: (i, 0))],
        out_specs=[
            pl.BlockSpec((rb, D), lambda i: (i, 0)),
            pl.BlockSpec((rb, D), lambda i: (i, 0)),
        ],
        out_shape=(
            jax.ShapeDtypeStruct((N, D), jnp.float32),
            jax.ShapeDtypeStruct((N, D), jnp.float32),
        ),
    )(x)
    nflat = nb[:, 0]

    edge_fn = _make_edge_kernel(e_full, n_blk, cpt)
    acc, denp = edge_fn(xn, nflat, src_full, dst_full, beta16)
    den_t = denp.reshape(NW, NP)[:, :N].T

    out = pl.pallas_call(
        _final_body,
        grid=(grid_r,),
        in_specs=[
            pl.BlockSpec((rb, D), lambda i: (i, 0)),
            pl.BlockSpec((rb, D), lambda i: (i, 0)),
            pl.BlockSpec((rb, NW), lambda i: (i, 0)),
        ],
        out_specs=pl.BlockSpec((rb, D), lambda i: (i, 0)),
        out_shape=jax.ShapeDtypeStruct((N, D), jnp.float32),
    )(acc[0, :N], acc[1, :N], den_t)
    return out


# speedup vs baseline: 17.6363x; 1.9236x over previous
import functools

import jax
import jax.numpy as jnp
from jax import lax
from jax.experimental import pallas as pl
from jax.experimental.pallas import tpu as pltpu
from jax.experimental.pallas import tpu_sc as plsc

N = 10000
D = 128
NC = 2
NS = 16
NW = NC * NS
B = 32
NP = 10240
ROWS_PER_TILE = NP // NS
DEN_R = NP // 128


def _norm_body(x_ref, xn_ref, nb_ref):
    xb = x_ref[...]
    ss = jnp.sum(xb * xb, axis=1, keepdims=True)
    nrm = jnp.maximum(jnp.sqrt(ss), 1e-12)
    xn_ref[...] = xb / nrm
    nb_ref[...] = jnp.broadcast_to(nrm, xb.shape)


def _final_body(a0_ref, a1_ref, dt_ref, o_ref):
    den = jnp.sum(dt_ref[...], axis=1, keepdims=True)
    s = a0_ref[...] + a1_ref[...]
    o_ref[...] = jnp.maximum(s / jnp.maximum(den, 1e-16), 0.0)


def _make_edge_kernel(e_full, n_blk, cpt):
    mesh = plsc.VectorSubcoreMesh(
        core_axis_name="c", subcore_axis_name="s", num_cores=NC, num_subcores=NS
    )

    @functools.partial(
        pl.kernel,
        out_type=(
            jax.ShapeDtypeStruct((NC, NP, D), jnp.float32),
            jax.ShapeDtypeStruct((NW, DEN_R, 128), jnp.float32),
        ),
        mesh=mesh,
        compiler_params=pltpu.CompilerParams(needs_layout_passes=False),
        scratch_types=[
            pltpu.VMEM((N,), jnp.float32),
            pltpu.VMEM((DEN_R, 128), jnp.float32),
            pltpu.VMEM((3, B), jnp.int32),
            pltpu.VMEM((3, B), jnp.int32),
            pltpu.VMEM((3, B), jnp.int32),
            pltpu.VMEM((3, B, D), jnp.float32),
            pltpu.VMEM((3, B, D), jnp.float32),
            pltpu.VMEM((16,), jnp.float32),
            pltpu.SemaphoreType.DMA((3,)),
            pltpu.SemaphoreType.DMA((3,)),
            pltpu.SemaphoreType.DMA((3,)),
            pltpu.VMEM_SHARED((NP, D), jnp.float32),
        ],
    )
    def edge_kernel(xn_hbm, n_hbm, src_hbm, dst_hbm, beta_hbm,
                    acc_out, den_out,
                    ntab, denom_v, sidx, didx, sdidx, srows, drows,
                    beta_v, isem, gsem, ssem, acc_sh):
        cid = lax.axis_index("c")
        sid = lax.axis_index("s")
        wid = cid * NS + sid

        pltpu.sync_copy(n_hbm, ntab)
        pltpu.sync_copy(beta_hbm, beta_v)
        bvec = beta_v[pl.ds(0, 16)]

        zero16 = jnp.zeros((16,), jnp.float32)

        def zero_den(i, carry):
            for s in range(8):
                denom_v[i, pl.ds(s * 16, 16)] = zero16
            return carry

        lax.fori_loop(0, DEN_R, zero_den, 0)

        def zero_sr(i, carry):
            for s in range(8):
                srows[0, i, pl.ds(s * 16, 16)] = zero16
            return carry

        lax.fori_loop(0, B, zero_sr, 0)

        row0 = sid * ROWS_PER_TILE
        for q in range(ROWS_PER_TILE // B):
            pltpu.sync_copy(srows.at[0], acc_sh.at[pl.ds(row0 + q * B, B)])
        plsc.subcore_barrier()

        lane = lax.iota(jnp.int32, 16)

        def idx_start(slot, blk_id):
            base = wid * cpt + blk_id * B
            pltpu.async_copy(src_hbm.at[pl.ds(base, B)], sidx.at[slot],
                             isem.at[slot])
            pltpu.async_copy(dst_hbm.at[pl.ds(base, B)], didx.at[slot],
                             isem.at[slot])

        def idx_wait(slot, blk_id):
            base = wid * cpt + blk_id * B
            pltpu.make_async_copy(src_hbm.at[pl.ds(base, B)], sidx.at[slot],
                                  isem.at[slot]).wait()
            pltpu.make_async_copy(dst_hbm.at[pl.ds(base, B)], didx.at[slot],
                                  isem.at[slot]).wait()

        def gather_start(slot):
            pltpu.async_copy(xn_hbm.at[sidx.at[slot]], srows.at[slot],
                             gsem.at[slot])
            pltpu.async_copy(xn_hbm.at[didx.at[slot]], drows.at[slot],
                             gsem.at[slot])

        def gather_wait(slot):
            pltpu.make_async_copy(xn_hbm.at[sidx.at[slot]], srows.at[slot],
                                  gsem.at[slot]).wait()
            pltpu.make_async_copy(xn_hbm.at[didx.at[slot]], drows.at[slot],
                                  gsem.at[slot]).wait()

        def scatter_start(slot):
            pltpu.async_copy(srows.at[slot], acc_sh.at[sdidx.at[slot]],
                             ssem.at[slot], add=True)

        def scatter_wait(slot):
            pltpu.make_async_copy(srows.at[slot], acc_sh.at[sdidx.at[slot]],
                                  ssem.at[slot]).wait()

        def compute_block(slot, g):
            base = wid * cpt + g * B

            def grp(k, c2):
                sl = pl.ds(k * 16, 16)
                si = sidx[slot, sl]
                di = didx[slot, sl]
                d16 = jnp.zeros((16,), jnp.float32)
                for j in range(16):
                    e = k * 16 + j
                    acc = (srows[slot, e, pl.ds(0, 16)]
                           * drows[slot, e, pl.ds(0, 16)])
                    for s in range(1, 8):
                        fsl = pl.ds(s * 16, 16)
                        acc = acc + srows[slot, e, fsl] * drows[slot, e, fsl]
                    d16 = jnp.where(lane == j, jnp.sum(acc), d16)
                gid = base + k * 16 + lane
                w = jnp.where(gid < e_full, jnp.exp(bvec * d16), 0.0)
                plsc.addupdate_scatter(denom_v, [di // 128, di % 128], w)
                n_s = plsc.load_gather(ntab, [si])
                wn = w * n_s
                sdidx[slot, sl] = di
                for j in range(16):
                    e = k * 16 + j
                    wj = wn[j]
                    for s in range(8):
                        fsl = pl.ds(s * 16, 16)
                        srows[slot, e, fsl] = srows[slot, e, fsl] * wj
                return c2

            lax.fori_loop(0, B // 16, grp, 0)

        idx_start(0, 0)
        idx_start(1, 1)
        idx_wait(0, 0)
        gather_start(0)

        def blk(g, carry):
            slot = lax.rem(g, 3)
            sn = lax.rem(g + 1, 3)
            sp = lax.rem(g + 2, 3)

            @pl.when(g >= 2)
            def _():
                scatter_wait(sn)

            @pl.when(g + 1 < n_blk)
            def _():
                idx_wait(sn, g + 1)
                gather_start(sn)

            @pl.when(g + 2 < n_blk)
            def _():
                idx_start(sp, g + 2)

            gather_wait(slot)
            compute_block(slot, g)
            scatter_start(slot)
            return carry

        lax.fori_loop(0, n_blk, blk, 0)
        scatter_wait((n_blk - 2) % 3)
        scatter_wait((n_blk - 1) % 3)

        plsc.subcore_barrier()
        pltpu.sync_copy(acc_sh.at[pl.ds(row0, ROWS_PER_TILE)],
                        acc_out.at[cid, pl.ds(row0, ROWS_PER_TILE)])
        pltpu.sync_copy(denom_v, den_out.at[wid])

    return edge_kernel


def kernel(x, edge_index, beta):
    src = edge_index[0].astype(jnp.int32)
    dst = edge_index[1].astype(jnp.int32)
    loop = jnp.arange(N, dtype=jnp.int32)
    e_full = src.shape[0] + N
    n_blk = -(-e_full // (NW * B))
    cpt = n_blk * B
    e_pad = NW * cpt
    pad = e_pad - e_full
    src_full = jnp.concatenate([src, loop, jnp.zeros((pad,), jnp.int32)])
    dst_full = jnp.concatenate([dst, loop, jnp.zeros((pad,), jnp.int32)])
    beta16 = jnp.broadcast_to(beta.astype(jnp.float32), (16,))

    grid_r = 10
    rb = N // grid_r
    xn, nb = pl.pallas_call(
        _norm_body,
        grid=(grid_r,),
        in_specs=[pl.BlockSpec((rb, D), lambda i: (i, 0))],
        out_specs=[
            pl.BlockSpec((rb, D), lambda i: (i, 0)),
            pl.BlockSpec((rb, D), lambda i: (i, 0)),
        ],
        out_shape=(
            jax.ShapeDtypeStruct((N, D), jnp.float32),
            jax.ShapeDtypeStruct((N, D), jnp.float32),
        ),
    )(x)
    nflat = nb[:, 0]

    edge_fn = _make_edge_kernel(e_full, n_blk, cpt)
    acc, denp = edge_fn(xn, nflat, src_full, dst_full, beta16)
    den_t = denp.reshape(NW, NP)[:, :N].T

    out = pl.pallas_call(
        _final_body,
        grid=(grid_r,),
        in_specs=[
            pl.BlockSpec((rb, D), lambda i: (i, 0)),
            pl.BlockSpec((rb, D), lambda i: (i, 0)),
            pl.BlockSpec((rb, NW), lambda i: (i, 0)),
        ],
        out_specs=pl.BlockSpec((rb, D), lambda i: (i, 0)),
        out_shape=jax.ShapeDtypeStruct((N, D), jnp.float32),
    )(acc[0, :N], acc[1, :N], den_t)
    return out


# speedup vs baseline: 18.1076x; 1.0267x over previous
import functools

import jax
import jax.numpy as jnp
from jax import lax
from jax.experimental import pallas as pl
from jax.experimental.pallas import tpu as pltpu
from jax.experimental.pallas import tpu_sc as plsc

N = 10000
D = 128
NC = 2
NS = 16
NW = NC * NS
B = 32
NP = 10240
ROWS_PER_TILE = NP // NS
DEN_R = NP // 128


def _norm_body(x_ref, xn_ref, nb_ref):
    xb = x_ref[...]
    ss = jnp.sum(xb * xb, axis=1, keepdims=True)
    nrm = jnp.maximum(jnp.sqrt(ss), 1e-12)
    xn_ref[...] = xb / nrm
    nb_ref[...] = jnp.broadcast_to(nrm, xb.shape)


def _final_body(a0_ref, a1_ref, dt_ref, o_ref):
    den = jnp.sum(dt_ref[...], axis=1, keepdims=True)
    s = a0_ref[0] + a1_ref[0]
    o_ref[...] = jnp.maximum(s / jnp.maximum(den, 1e-16), 0.0)


def _make_edge_kernel(e_full, n_blk, cpt):
    mesh = plsc.VectorSubcoreMesh(
        core_axis_name="c", subcore_axis_name="s", num_cores=NC, num_subcores=NS
    )

    @functools.partial(
        pl.kernel,
        out_type=(
            jax.ShapeDtypeStruct((NC, NP, D), jnp.float32),
            jax.ShapeDtypeStruct((NC, DEN_R, 128), jnp.float32),
        ),
        mesh=mesh,
        compiler_params=pltpu.CompilerParams(needs_layout_passes=False),
        scratch_types=[
            pltpu.VMEM((N,), jnp.float32),
            pltpu.VMEM((DEN_R, 128), jnp.float32),
            pltpu.VMEM((3, B), jnp.int32),
            pltpu.VMEM((3, B), jnp.int32),
            pltpu.VMEM((3, B), jnp.int32),
            pltpu.VMEM((3, B, D), jnp.float32),
            pltpu.VMEM((3, B, D), jnp.float32),
            pltpu.VMEM((16,), jnp.float32),
            pltpu.VMEM((DEN_R,), jnp.int32),
            pltpu.SemaphoreType.DMA((3,)),
            pltpu.SemaphoreType.DMA((3,)),
            pltpu.SemaphoreType.DMA((3,)),
            pltpu.VMEM_SHARED((NP, D), jnp.float32),
            pltpu.VMEM_SHARED((DEN_R, 128), jnp.float32),
        ],
    )
    def edge_kernel(xn_hbm, n_hbm, src_hbm, dst_hbm, beta_hbm,
                    acc_out, den_out,
                    ntab, denom_v, sidx, didx, sdidx, srows, drows,
                    beta_v, iden, isem, gsem, ssem, acc_sh, dden_sh):
        cid = lax.axis_index("c")
        sid = lax.axis_index("s")
        wid = cid * NS + sid

        pltpu.sync_copy(n_hbm, ntab)
        pltpu.sync_copy(beta_hbm, beta_v)
        bvec = beta_v[pl.ds(0, 16)]

        zero16 = jnp.zeros((16,), jnp.float32)

        def zero_den(i, carry):
            for s in range(8):
                denom_v[i, pl.ds(s * 16, 16)] = zero16
            return carry

        lax.fori_loop(0, DEN_R, zero_den, 0)

        def zero_sr(i, carry):
            for s in range(8):
                srows[0, i, pl.ds(s * 16, 16)] = zero16
            return carry

        lax.fori_loop(0, B, zero_sr, 0)

        lane = lax.iota(jnp.int32, 16)
        for q in range(DEN_R // 16):
            iden[pl.ds(q * 16, 16)] = lane + q * 16

        row0 = sid * ROWS_PER_TILE
        for q in range(ROWS_PER_TILE // B):
            pltpu.sync_copy(srows.at[0], acc_sh.at[pl.ds(row0 + q * B, B)])

        @pl.when(sid == 0)
        def _():
            pltpu.sync_copy(srows.at[0], dden_sh.at[pl.ds(0, B)])
            pltpu.sync_copy(srows.at[0], dden_sh.at[pl.ds(B, B)])
            pltpu.sync_copy(srows.at[0, pl.ds(0, DEN_R - 2 * B)],
                            dden_sh.at[pl.ds(2 * B, DEN_R - 2 * B)])

        plsc.subcore_barrier()

        def idx_start(slot, blk_id):
            base = wid * cpt + blk_id * B
            pltpu.async_copy(src_hbm.at[pl.ds(base, B)], sidx.at[slot],
                             isem.at[slot])
            pltpu.async_copy(dst_hbm.at[pl.ds(base, B)], didx.at[slot],
                             isem.at[slot])

        def idx_wait(slot, blk_id):
            base = wid * cpt + blk_id * B
            pltpu.make_async_copy(src_hbm.at[pl.ds(base, B)], sidx.at[slot],
                                  isem.at[slot]).wait()
            pltpu.make_async_copy(dst_hbm.at[pl.ds(base, B)], didx.at[slot],
                                  isem.at[slot]).wait()

        def gather_start(slot):
            pltpu.async_copy(xn_hbm.at[sidx.at[slot]], srows.at[slot],
                             gsem.at[slot])
            pltpu.async_copy(xn_hbm.at[didx.at[slot]], drows.at[slot],
                             gsem.at[slot])

        def gather_wait(slot):
            pltpu.make_async_copy(xn_hbm.at[sidx.at[slot]], srows.at[slot],
                                  gsem.at[slot]).wait()
            pltpu.make_async_copy(xn_hbm.at[didx.at[slot]], drows.at[slot],
                                  gsem.at[slot]).wait()

        def scatter_start(slot):
            pltpu.async_copy(srows.at[slot], acc_sh.at[sdidx.at[slot]],
                             ssem.at[slot], add=True)

        def scatter_wait(slot):
            pltpu.make_async_copy(srows.at[slot], acc_sh.at[sdidx.at[slot]],
                                  ssem.at[slot]).wait()

        def compute_block(slot, g):
            base = wid * cpt + g * B

            def grp(k, c2):
                sl = pl.ds(k * 16, 16)
                si = sidx[slot, sl]
                di = didx[slot, sl]
                d16 = jnp.zeros((16,), jnp.float32)
                for j in range(16):
                    e = k * 16 + j
                    acc = (srows[slot, e, pl.ds(0, 16)]
                           * drows[slot, e, pl.ds(0, 16)])
                    for s in range(1, 8):
                        fsl = pl.ds(s * 16, 16)
                        acc = acc + srows[slot, e, fsl] * drows[slot, e, fsl]
                    d16 = jnp.where(lane == j, jnp.sum(acc), d16)
                gid = base + k * 16 + lane
                w = jnp.where(gid < e_full, jnp.exp(bvec * d16), 0.0)
                plsc.addupdate_scatter(denom_v, [di // 128, di % 128], w)
                n_s = plsc.load_gather(ntab, [si])
                wn = w * n_s
                sdidx[slot, sl] = di
                for j in range(16):
                    e = k * 16 + j
                    wj = wn[j]
                    for s in range(8):
                        fsl = pl.ds(s * 16, 16)
                        srows[slot, e, fsl] = srows[slot, e, fsl] * wj
                return c2

            lax.fori_loop(0, B // 16, grp, 0)

        idx_start(0, 0)
        idx_start(1, 1)
        idx_wait(0, 0)
        gather_start(0)

        def blk(g, carry):
            slot = lax.rem(g, 3)
            sn = lax.rem(g + 1, 3)
            sp = lax.rem(g + 2, 3)

            @pl.when(g >= 2)
            def _():
                scatter_wait(sn)

            @pl.when(g + 1 < n_blk)
            def _():
                idx_wait(sn, g + 1)
                gather_start(sn)

            @pl.when(g + 2 < n_blk)
            def _():
                idx_start(sp, g + 2)

            gather_wait(slot)
            compute_block(slot, g)
            scatter_start(slot)
            return carry

        lax.fori_loop(0, n_blk, blk, 0)
        scatter_wait((n_blk - 2) % 3)
        scatter_wait((n_blk - 1) % 3)

        pltpu.sync_copy(denom_v, dden_sh.at[iden], add=True)
        plsc.subcore_barrier()
        pltpu.sync_copy(acc_sh.at[pl.ds(row0, ROWS_PER_TILE)],
                        acc_out.at[cid, pl.ds(row0, ROWS_PER_TILE)])

        @pl.when(sid == 0)
        def _():
            pltpu.sync_copy(dden_sh, den_out.at[cid])

    return edge_kernel


def kernel(x, edge_index, beta):
    src = edge_index[0].astype(jnp.int32)
    dst = edge_index[1].astype(jnp.int32)
    loop = jnp.arange(N, dtype=jnp.int32)
    e_full = src.shape[0] + N
    n_blk = -(-e_full // (NW * B))
    cpt = n_blk * B
    e_pad = NW * cpt
    pad = e_pad - e_full
    src_full = jnp.concatenate([src, loop, jnp.zeros((pad,), jnp.int32)])
    dst_full = jnp.concatenate([dst, loop, jnp.zeros((pad,), jnp.int32)])
    beta16 = jnp.broadcast_to(beta.astype(jnp.float32), (16,))

    grid_r = 10
    rb = N // grid_r
    xn, nb = pl.pallas_call(
        _norm_body,
        grid=(grid_r,),
        in_specs=[pl.BlockSpec((rb, D), lambda i: (i, 0))],
        out_specs=[
            pl.BlockSpec((rb, D), lambda i: (i, 0)),
            pl.BlockSpec((rb, D), lambda i: (i, 0)),
        ],
        out_shape=(
            jax.ShapeDtypeStruct((N, D), jnp.float32),
            jax.ShapeDtypeStruct((N, D), jnp.float32),
        ),
    )(x)
    nflat = nb[:, 0]

    edge_fn = _make_edge_kernel(e_full, n_blk, cpt)
    acc, denp = edge_fn(xn, nflat, src_full, dst_full, beta16)
    den_t = denp.reshape(NC, NP)[:, :N].T

    out = pl.pallas_call(
        _final_body,
        grid=(grid_r,),
        in_specs=[
            pl.BlockSpec((1, rb, D), lambda i: (0, i, 0)),
            pl.BlockSpec((1, rb, D), lambda i: (1, i, 0)),
            pl.BlockSpec((rb, NC), lambda i: (i, 0)),
        ],
        out_specs=pl.BlockSpec((rb, D), lambda i: (i, 0)),
        out_shape=jax.ShapeDtypeStruct((N, D), jnp.float32),
    )(acc, acc, den_t)
    return out
